# Initial kernel scaffold; baseline (speedup 1.0000x reference)
import functools

import jax
import jax.numpy as jnp
from jax import lax
from jax.experimental import pallas as pl
from jax.experimental.pallas import tpu as pltpu
from jax.experimental.pallas import tpu_sc as plsc

N_NODES = 10000
N_EDGES = 320000
D = 128
LANES = 16

NC, NS = 2, 16
NW = NC * NS
E_PER_W = N_EDGES // NW
K = 80
CHUNKS = E_PER_W // K
ROWS_PER_TILE = N_NODES // NS

NODE_BLK = 1000


def _precompute_body(h_ref, w_ref, b_ref, a_ref, bm_ref):
    hb = h_ref[...]
    w = w_ref[...]
    a_ref[...] = jnp.dot(hb, w[:D], preferred_element_type=jnp.float32)
    bm_ref[...] = (
        jnp.dot(hb, w[D:], preferred_element_type=jnp.float32) + b_ref[...]
    )


def _precompute(h, W_msg, b_msg2d):
    grid = N_NODES // NODE_BLK
    return pl.pallas_call(
        _precompute_body,
        grid=(grid,),
        in_specs=[
            pl.BlockSpec((NODE_BLK, D), lambda i: (i, 0)),
            pl.BlockSpec((2 * D, D), lambda i: (0, 0)),
            pl.BlockSpec((1, D), lambda i: (0, 0)),
        ],
        out_specs=[
            pl.BlockSpec((NODE_BLK, D), lambda i: (i, 0)),
            pl.BlockSpec((NODE_BLK, D), lambda i: (i, 0)),
        ],
        out_shape=[
            jax.ShapeDtypeStruct((N_NODES, D), jnp.float32),
            jax.ShapeDtypeStruct((N_NODES, D), jnp.float32),
        ],
    )(h, W_msg, b_msg2d)


def _node_update_body(h_ref, m0_ref, m1_ref, w_ref, b_ref, o_ref):
    w = w_ref[...]
    m = m0_ref[...] + m1_ref[...]
    acc = (
        jnp.dot(h_ref[...], w[:D], preferred_element_type=jnp.float32)
        + jnp.dot(m, w[D:], preferred_element_type=jnp.float32)
        + b_ref[...]
    )
    o_ref[...] = jnp.maximum(acc, 0.0)


def _node_update(h, m0, m1, W_node, b_node2d):
    grid = N_NODES // NODE_BLK
    return pl.pallas_call(
        _node_update_body,
        grid=(grid,),
        in_specs=[
            pl.BlockSpec((NODE_BLK, D), lambda i: (i, 0)),
            pl.BlockSpec((NODE_BLK, D), lambda i: (i, 0)),
            pl.BlockSpec((NODE_BLK, D), lambda i: (i, 0)),
            pl.BlockSpec((2 * D, D), lambda i: (0, 0)),
            pl.BlockSpec((1, D), lambda i: (0, 0)),
        ],
        out_specs=pl.BlockSpec((NODE_BLK, D), lambda i: (i, 0)),
        out_shape=jax.ShapeDtypeStruct((N_NODES, D), jnp.float32),
    )(h, m0, m1, W_node, b_node2d)


@functools.partial(
    pl.kernel,
    out_type=jax.ShapeDtypeStruct((NC * N_NODES, D), jnp.float32),
    mesh=plsc.VectorSubcoreMesh(core_axis_name="c", subcore_axis_name="s"),
    scratch_types=[
        pltpu.VMEM((K,), jnp.int32),
        pltpu.VMEM((K,), jnp.int32),
        pltpu.VMEM((K, D), jnp.float32),
        pltpu.VMEM((K, D), jnp.float32),
        pltpu.VMEM_SHARED((N_NODES, D), jnp.float32),
        pltpu.SemaphoreType.DMA,
        pltpu.SemaphoreType.DMA,
    ],
)
def _sc_edge_kernel(a_hbm, b_hbm, src_hbm, dst_hbm, zeros_hbm, out_hbm,
                    idx_s, idx_d, buf_a, buf_b, acc, sem_a, sem_b):
    c = lax.axis_index("c")
    s = lax.axis_index("s")
    wid = s * NC + c
    base = wid * E_PER_W

    pltpu.sync_copy(
        zeros_hbm.at[pl.ds(s * ROWS_PER_TILE, ROWS_PER_TILE)],
        acc.at[pl.ds(s * ROWS_PER_TILE, ROWS_PER_TILE)],
    )
    plsc.subcore_barrier()

    def chunk_body(i, carry):
        off = base + i * K
        pltpu.sync_copy(src_hbm.at[pl.ds(off, K)], idx_s)
        pltpu.sync_copy(dst_hbm.at[pl.ds(off, K)], idx_d)
        ca = pltpu.async_copy(a_hbm.at[idx_s], buf_a, sem_a)
        cb = pltpu.async_copy(b_hbm.at[idx_d], buf_b, sem_b)
        ca.wait()
        cb.wait()

        def row_body(r, rc):
            for g in range(D // LANES):
                sl = pl.ds(g * LANES, LANES)
                buf_a[r, sl] = jnp.maximum(buf_a[r, sl] + buf_b[r, sl], 0.0)
            return rc

        lax.fori_loop(0, K, row_body, 0)
        pltpu.sync_copy(buf_a, acc.at[idx_d], add=True)
        return carry

    lax.fori_loop(0, CHUNKS, chunk_body, 0)

    plsc.subcore_barrier()
    pltpu.sync_copy(
        acc.at[pl.ds(s * ROWS_PER_TILE, ROWS_PER_TILE)],
        out_hbm.at[pl.ds(c * N_NODES + s * ROWS_PER_TILE, ROWS_PER_TILE)],
    )


def kernel(h, edge_index, W_msg, b_msg, W_node, b_node):
    src = edge_index[0].astype(jnp.int32)
    dst = edge_index[1].astype(jnp.int32)
    a, b = _precompute(h, W_msg, b_msg.reshape(1, D))
    zeros = jnp.zeros((N_NODES, D), jnp.float32)
    mp = _sc_edge_kernel(a, b, src, dst, zeros)
    return _node_update(h, mp[:N_NODES], mp[N_NODES:], W_node,
                        b_node.reshape(1, D))


# speedup vs baseline: 5.6409x; 5.6409x over previous
import functools

import jax
import jax.numpy as jnp
from jax import lax
from jax.experimental import pallas as pl
from jax.experimental.pallas import tpu as pltpu
from jax.experimental.pallas import tpu_sc as plsc

N_NODES = 10000
N_EDGES = 320000
D = 128
LANES = 16

NC, NS = 2, 16
NW = NC * NS
E_PER_W = N_EDGES // NW
K = 80
CHUNKS = E_PER_W // K
STRIPE = 640
STRIDE = 624

NODE_BLK = 1000


def _precompute_body(h_ref, w_ref, b_ref, a_ref, bm_ref):
    hb = h_ref[...]
    w = w_ref[...]
    a_ref[...] = jnp.dot(hb, w[:D], preferred_element_type=jnp.float32)
    bm_ref[...] = (
        jnp.dot(hb, w[D:], preferred_element_type=jnp.float32) + b_ref[...]
    )


def _precompute(h, W_msg, b_msg2d):
    grid = N_NODES // NODE_BLK
    return pl.pallas_call(
        _precompute_body,
        grid=(grid,),
        in_specs=[
            pl.BlockSpec((NODE_BLK, D), lambda i: (i, 0)),
            pl.BlockSpec((2 * D, D), lambda i: (0, 0)),
            pl.BlockSpec((1, D), lambda i: (0, 0)),
        ],
        out_specs=[
            pl.BlockSpec((NODE_BLK, D), lambda i: (i, 0)),
            pl.BlockSpec((NODE_BLK, D), lambda i: (i, 0)),
        ],
        out_shape=[
            jax.ShapeDtypeStruct((N_NODES, D), jnp.float32),
            jax.ShapeDtypeStruct((N_NODES, D), jnp.float32),
        ],
    )(h, W_msg, b_msg2d)


def _node_update_body(h_ref, m0_ref, m1_ref, w_ref, b_ref, o_ref):
    w = w_ref[...]
    m = m0_ref[...] + m1_ref[...]
    acc = (
        jnp.dot(h_ref[...], w[:D], preferred_element_type=jnp.float32)
        + jnp.dot(m, w[D:], preferred_element_type=jnp.float32)
        + b_ref[...]
    )
    o_ref[...] = jnp.maximum(acc, 0.0)


def _node_update(h, m0, m1, W_node, b_node2d):
    grid = N_NODES // NODE_BLK
    return pl.pallas_call(
        _node_update_body,
        grid=(grid,),
        in_specs=[
            pl.BlockSpec((NODE_BLK, D), lambda i: (i, 0)),
            pl.BlockSpec((NODE_BLK, D), lambda i: (i, 0)),
            pl.BlockSpec((NODE_BLK, D), lambda i: (i, 0)),
            pl.BlockSpec((2 * D, D), lambda i: (0, 0)),
            pl.BlockSpec((1, D), lambda i: (0, 0)),
        ],
        out_specs=pl.BlockSpec((NODE_BLK, D), lambda i: (i, 0)),
        out_shape=jax.ShapeDtypeStruct((N_NODES, D), jnp.float32),
    )(h, m0, m1, W_node, b_node2d)


@functools.partial(
    pl.kernel,
    out_type=jax.ShapeDtypeStruct((NC * N_NODES, D), jnp.float32),
    mesh=plsc.VectorSubcoreMesh(core_axis_name="c", subcore_axis_name="s"),
    scratch_types=[
        pltpu.VMEM((K,), jnp.int32),
        pltpu.VMEM((K,), jnp.int32),
        pltpu.VMEM((K, D), jnp.float32),
        pltpu.VMEM((K, D), jnp.float32),
        pltpu.VMEM_SHARED((N_NODES, D), jnp.float32),
        pltpu.SemaphoreType.DMA,
        pltpu.SemaphoreType.DMA,
    ],
)
def _sc_edge_kernel(a_hbm, b_hbm, src_hbm, dst_hbm, zeros_hbm, out_hbm,
                    idx_s, idx_d, buf_a, buf_b, acc, sem_a, sem_b):
    c = lax.axis_index("c")
    s = lax.axis_index("s")
    wid = s * NC + c
    base = wid * E_PER_W

    pltpu.sync_copy(
        zeros_hbm.at[pl.ds(s * STRIDE, STRIPE)],
        acc.at[pl.ds(s * STRIDE, STRIPE)],
    )
    plsc.subcore_barrier()

    def chunk_body(i, carry):
        off = base + i * K
        pltpu.sync_copy(src_hbm.at[pl.ds(off, K)], idx_s)
        pltpu.sync_copy(dst_hbm.at[pl.ds(off, K)], idx_d)
        ca = pltpu.async_copy(a_hbm.at[idx_s], buf_a, sem_a)
        cb = pltpu.async_copy(b_hbm.at[idx_d], buf_b, sem_b)
        ca.wait()
        cb.wait()

        def row_body(r, rc):
            for g in range(D // LANES):
                sl = pl.ds(g * LANES, LANES)
                buf_a[r, sl] = jnp.maximum(buf_a[r, sl] + buf_b[r, sl], 0.0)
            return rc

        lax.fori_loop(0, K, row_body, 0)
        pltpu.sync_copy(buf_a, acc.at[idx_d], add=True)
        return carry

    lax.fori_loop(0, CHUNKS, chunk_body, 0)

    plsc.subcore_barrier()
    pltpu.sync_copy(
        acc.at[pl.ds(s * STRIDE, STRIPE)],
        out_hbm.at[pl.ds(c * N_NODES + s * STRIDE, STRIPE)],
    )


def kernel(h, edge_index, W_msg, b_msg, W_node, b_node):
    src = edge_index[0].astype(jnp.int32)
    dst = edge_index[1].astype(jnp.int32)
    a, b = _precompute(h, W_msg, b_msg.reshape(1, D))
    zeros = jnp.zeros((N_NODES, D), jnp.float32)
    mp = _sc_edge_kernel(a, b, src, dst, zeros)
    return _node_update(h, mp[:N_NODES], mp[N_NODES:], W_node,
                        b_node.reshape(1, D))


# speedup vs baseline: 9.8455x; 1.7454x over previous
import functools

import jax
import jax.numpy as jnp
from jax import lax
from jax.experimental import pallas as pl
from jax.experimental.pallas import tpu as pltpu
from jax.experimental.pallas import tpu_sc as plsc

N_NODES = 10000
N_EDGES = 320000
D = 128
LANES = 16

NC, NS = 2, 16
NW = NC * NS
E_PER_W = N_EDGES // NW
K = 80
CHUNKS = E_PER_W // K
STRIPE = 640
STRIDE = 624

NODE_BLK = 1000


def _precompute_body(h_ref, w_ref, b_ref, a_ref, bm_ref):
    hb = h_ref[...]
    w = w_ref[...]
    a_ref[...] = jnp.dot(hb, w[:D], preferred_element_type=jnp.float32)
    bm_ref[...] = (
        jnp.dot(hb, w[D:], preferred_element_type=jnp.float32) + b_ref[...]
    )


def _precompute(h, W_msg, b_msg2d):
    grid = N_NODES // NODE_BLK
    return pl.pallas_call(
        _precompute_body,
        grid=(grid,),
        in_specs=[
            pl.BlockSpec((NODE_BLK, D), lambda i: (i, 0)),
            pl.BlockSpec((2 * D, D), lambda i: (0, 0)),
            pl.BlockSpec((1, D), lambda i: (0, 0)),
        ],
        out_specs=[
            pl.BlockSpec((NODE_BLK, D), lambda i: (i, 0)),
            pl.BlockSpec((NODE_BLK, D), lambda i: (i, 0)),
        ],
        out_shape=[
            jax.ShapeDtypeStruct((N_NODES, D), jnp.float32),
            jax.ShapeDtypeStruct((N_NODES, D), jnp.float32),
        ],
    )(h, W_msg, b_msg2d)


def _node_update_body(h_ref, m0_ref, m1_ref, w_ref, b_ref, o_ref):
    w = w_ref[...]
    m = m0_ref[...] + m1_ref[...]
    acc = (
        jnp.dot(h_ref[...], w[:D], preferred_element_type=jnp.float32)
        + jnp.dot(m, w[D:], preferred_element_type=jnp.float32)
        + b_ref[...]
    )
    o_ref[...] = jnp.maximum(acc, 0.0)


def _node_update(h, m0, m1, W_node, b_node2d):
    grid = N_NODES // NODE_BLK
    return pl.pallas_call(
        _node_update_body,
        grid=(grid,),
        in_specs=[
            pl.BlockSpec((NODE_BLK, D), lambda i: (i, 0)),
            pl.BlockSpec((NODE_BLK, D), lambda i: (i, 0)),
            pl.BlockSpec((NODE_BLK, D), lambda i: (i, 0)),
            pl.BlockSpec((2 * D, D), lambda i: (0, 0)),
            pl.BlockSpec((1, D), lambda i: (0, 0)),
        ],
        out_specs=pl.BlockSpec((NODE_BLK, D), lambda i: (i, 0)),
        out_shape=jax.ShapeDtypeStruct((N_NODES, D), jnp.float32),
    )(h, m0, m1, W_node, b_node2d)


@functools.partial(
    pl.kernel,
    out_type=jax.ShapeDtypeStruct((NC * N_NODES, D), jnp.float32),
    mesh=plsc.VectorSubcoreMesh(core_axis_name="c", subcore_axis_name="s"),
    scratch_types=[
        [pltpu.VMEM((K,), jnp.int32)] * 2,
        [pltpu.VMEM((K,), jnp.int32)] * 2,
        [pltpu.VMEM((K, D), jnp.float32)] * 2,
        [pltpu.VMEM((K, D), jnp.float32)] * 2,
        pltpu.VMEM_SHARED((N_NODES, D), jnp.float32),
        [pltpu.SemaphoreType.DMA] * 2,
        [pltpu.SemaphoreType.DMA] * 2,
        [pltpu.SemaphoreType.DMA] * 2,
    ],
)
def _sc_edge_kernel(a_hbm, b_hbm, src_hbm, dst_hbm, zeros_hbm, out_hbm,
                    idx_s, idx_d, buf_a, buf_b, acc, sem_i, sem_a, sem_b):
    c = lax.axis_index("c")
    s = lax.axis_index("s")
    wid = s * NC + c
    base = wid * E_PER_W

    pltpu.sync_copy(
        zeros_hbm.at[pl.ds(s * STRIDE, STRIPE)],
        acc.at[pl.ds(s * STRIDE, STRIPE)],
    )
    plsc.subcore_barrier()

    def fire_idx(j, p):
        off = base + j * K
        pltpu.async_copy(src_hbm.at[pl.ds(off, K)], idx_s[p], sem_i[p])
        pltpu.async_copy(dst_hbm.at[pl.ds(off, K)], idx_d[p], sem_i[p])

    def wait_idx(p):
        pltpu.make_async_copy(src_hbm.at[pl.ds(0, K)], idx_s[p], sem_i[p]).wait()
        pltpu.make_async_copy(dst_hbm.at[pl.ds(0, K)], idx_d[p], sem_i[p]).wait()

    def fire_gather(p):
        pltpu.async_copy(a_hbm.at[idx_s[p]], buf_a[p], sem_a[p])
        pltpu.async_copy(b_hbm.at[idx_d[p]], buf_b[p], sem_b[p])

    def wait_gather(p):
        pltpu.make_async_copy(a_hbm.at[idx_s[p]], buf_a[p], sem_a[p]).wait()
        pltpu.make_async_copy(b_hbm.at[idx_d[p]], buf_b[p], sem_b[p]).wait()

    def consume(p):
        def row_body(r, rc):
            for g in range(D // LANES):
                sl = pl.ds(g * LANES, LANES)
                buf_a[p][r, sl] = jnp.maximum(
                    buf_a[p][r, sl] + buf_b[p][r, sl], 0.0)
            return rc

        lax.fori_loop(0, K, row_body, 0)
        pltpu.sync_copy(buf_a[p], acc.at[idx_d[p]], add=True)

    fire_idx(0, 0)
    wait_idx(0)
    fire_gather(0)
    fire_idx(1, 1)

    def step(j, p):
        @pl.when(j + 1 < CHUNKS)
        def _():
            wait_idx(1 - p)
            fire_gather(1 - p)
        wait_gather(p)
        consume(p)

        @pl.when(j + 2 < CHUNKS)
        def _():
            fire_idx(j + 2, p)

    def pair_body(t, carry):
        step(2 * t, 0)

        @pl.when(2 * t + 1 < CHUNKS)
        def _():
            step(2 * t + 1, 1)
        return carry

    lax.fori_loop(0, (CHUNKS + 1) // 2, pair_body, 0)

    plsc.subcore_barrier()
    pltpu.sync_copy(
        acc.at[pl.ds(s * STRIDE, STRIPE)],
        out_hbm.at[pl.ds(c * N_NODES + s * STRIDE, STRIPE)],
    )


def kernel(h, edge_index, W_msg, b_msg, W_node, b_node):
    src = edge_index[0].astype(jnp.int32)
    dst = edge_index[1].astype(jnp.int32)
    a, b = _precompute(h, W_msg, b_msg.reshape(1, D))
    zeros = jnp.zeros((N_NODES, D), jnp.float32)
    mp = _sc_edge_kernel(a, b, src, dst, zeros)
    return _node_update(h, mp[:N_NODES], mp[N_NODES:], W_node,
                        b_node.reshape(1, D))


# speedup vs baseline: 10.7894x; 1.0959x over previous
import functools

import jax
import jax.numpy as jnp
from jax import lax
from jax.experimental import pallas as pl
from jax.experimental.pallas import tpu as pltpu
from jax.experimental.pallas import tpu_sc as plsc

N_NODES = 10000
N_EDGES = 320000
D = 128
LANES = 16

NC, NS = 2, 16
NW = NC * NS
E_PER_W = N_EDGES // NW
K = 40
CHUNKS = E_PER_W // K
STRIPE = 640
STRIDE = 624

NODE_BLK = 1000


def _precompute_body(h_ref, w_ref, b_ref, a_ref, bm_ref):
    hb = h_ref[...]
    w = w_ref[...]
    a_ref[...] = jnp.dot(hb, w[:D], preferred_element_type=jnp.float32)
    bm_ref[...] = (
        jnp.dot(hb, w[D:], preferred_element_type=jnp.float32) + b_ref[...]
    )


def _precompute(h, W_msg, b_msg2d):
    grid = N_NODES // NODE_BLK
    return pl.pallas_call(
        _precompute_body,
        grid=(grid,),
        in_specs=[
            pl.BlockSpec((NODE_BLK, D), lambda i: (i, 0)),
            pl.BlockSpec((2 * D, D), lambda i: (0, 0)),
            pl.BlockSpec((1, D), lambda i: (0, 0)),
        ],
        out_specs=[
            pl.BlockSpec((NODE_BLK, D), lambda i: (i, 0)),
            pl.BlockSpec((NODE_BLK, D), lambda i: (i, 0)),
        ],
        out_shape=[
            jax.ShapeDtypeStruct((N_NODES, D), jnp.float32),
            jax.ShapeDtypeStruct((N_NODES, D), jnp.float32),
        ],
    )(h, W_msg, b_msg2d)


def _node_update_body(h_ref, m0_ref, m1_ref, w_ref, b_ref, o_ref):
    w = w_ref[...]
    m = m0_ref[...] + m1_ref[...]
    acc = (
        jnp.dot(h_ref[...], w[:D], preferred_element_type=jnp.float32)
        + jnp.dot(m, w[D:], preferred_element_type=jnp.float32)
        + b_ref[...]
    )
    o_ref[...] = jnp.maximum(acc, 0.0)


def _node_update(h, m0, m1, W_node, b_node2d):
    grid = N_NODES // NODE_BLK
    return pl.pallas_call(
        _node_update_body,
        grid=(grid,),
        in_specs=[
            pl.BlockSpec((NODE_BLK, D), lambda i: (i, 0)),
            pl.BlockSpec((NODE_BLK, D), lambda i: (i, 0)),
            pl.BlockSpec((NODE_BLK, D), lambda i: (i, 0)),
            pl.BlockSpec((2 * D, D), lambda i: (0, 0)),
            pl.BlockSpec((1, D), lambda i: (0, 0)),
        ],
        out_specs=pl.BlockSpec((NODE_BLK, D), lambda i: (i, 0)),
        out_shape=jax.ShapeDtypeStruct((N_NODES, D), jnp.float32),
    )(h, m0, m1, W_node, b_node2d)


@functools.partial(
    pl.kernel,
    out_type=jax.ShapeDtypeStruct((NC * N_NODES, D), jnp.float32),
    mesh=plsc.VectorSubcoreMesh(core_axis_name="c", subcore_axis_name="s"),
    scratch_types=[
        [pltpu.VMEM((K,), jnp.int32)] * 4,
        [pltpu.VMEM((K,), jnp.int32)] * 4,
        [pltpu.VMEM((K, D), jnp.float32)] * 3,
        [pltpu.VMEM((K, D), jnp.float32)] * 3,
        pltpu.VMEM_SHARED((N_NODES, D), jnp.float32),
        [pltpu.SemaphoreType.DMA] * 4,
        [pltpu.SemaphoreType.DMA] * 3,
        [pltpu.SemaphoreType.DMA] * 3,
        [pltpu.SemaphoreType.DMA] * 3,
    ],
)
def _sc_edge_kernel(a_hbm, b_hbm, src_hbm, dst_hbm, zeros_hbm, out_hbm,
                    idx_s, idx_d, buf_a, buf_b, acc,
                    sem_i, sem_a, sem_b, sem_s):
    c = lax.axis_index("c")
    s = lax.axis_index("s")
    wid = s * NC + c
    base = wid * E_PER_W

    pltpu.sync_copy(
        zeros_hbm.at[pl.ds(s * STRIDE, STRIPE)],
        acc.at[pl.ds(s * STRIDE, STRIPE)],
    )
    plsc.subcore_barrier()

    def fire_idx(j, pi):
        off = base + j * K
        pltpu.async_copy(src_hbm.at[pl.ds(off, K)], idx_s[pi], sem_i[pi])
        pltpu.async_copy(dst_hbm.at[pl.ds(off, K)], idx_d[pi], sem_i[pi])

    def wait_idx(pi):
        pltpu.make_async_copy(src_hbm.at[pl.ds(0, K)], idx_s[pi], sem_i[pi]).wait()
        pltpu.make_async_copy(dst_hbm.at[pl.ds(0, K)], idx_d[pi], sem_i[pi]).wait()

    def fire_gather(pr, pi):
        pltpu.async_copy(a_hbm.at[idx_s[pi]], buf_a[pr], sem_a[pr])
        pltpu.async_copy(b_hbm.at[idx_d[pi]], buf_b[pr], sem_b[pr])

    def wait_gather(pr, pi):
        pltpu.make_async_copy(a_hbm.at[idx_s[pi]], buf_a[pr], sem_a[pr]).wait()
        pltpu.make_async_copy(b_hbm.at[idx_d[pi]], buf_b[pr], sem_b[pr]).wait()

    def compute(pr):
        def row_body(r, rc):
            for g in range(D // LANES):
                sl = pl.ds(g * LANES, LANES)
                buf_a[pr][r, sl] = jnp.maximum(
                    buf_a[pr][r, sl] + buf_b[pr][r, sl], 0.0)
            return rc

        lax.fori_loop(0, K, row_body, 0)

    def fire_scatter(pr, pi):
        pltpu.async_copy(buf_a[pr], acc.at[idx_d[pi]], sem_s[pr], add=True)

    def wait_scatter(pr, pi):
        pltpu.make_async_copy(buf_a[pr], acc.at[idx_d[pi]], sem_s[pr]).wait()

    fire_idx(0, 0)
    fire_idx(1, 1)
    fire_idx(2, 2)
    wait_idx(0)
    fire_gather(0, 0)
    wait_idx(1)
    fire_gather(1, 1)

    STEPS_MAIN = 12 * ((CHUNKS - 5) // 12)

    def block_body(t, carry):
        j0 = 12 * t
        for k in range(12):
            j = j0 + k
            jr, ji = k % 3, k % 4
            wait_gather(jr, ji)
            compute(jr)
            fire_scatter(jr, ji)
            if k == 0:
                @pl.when(t > 0)
                def _():
                    wait_scatter(2, 3)
            else:
                wait_scatter((k - 1) % 3, (k - 1) % 4)
            wait_idx((k + 2) % 4)
            fire_gather((k + 2) % 3, (k + 2) % 4)
            fire_idx(j + 3, (k + 3) % 4)
        return carry

    lax.fori_loop(0, STEPS_MAIN // 12, block_body, 0)

    for j in range(STEPS_MAIN, CHUNKS):
        jr, ji = j % 3, j % 4
        wait_gather(jr, ji)
        compute(jr)
        fire_scatter(jr, ji)
        wait_scatter((j - 1) % 3, (j - 1) % 4)
        if j + 2 < CHUNKS:
            wait_idx((j + 2) % 4)
            fire_gather((j + 2) % 3, (j + 2) % 4)
        if j + 3 < CHUNKS:
            fire_idx(j + 3, (j + 3) % 4)

    wait_scatter((CHUNKS - 1) % 3, (CHUNKS - 1) % 4)

    plsc.subcore_barrier()
    pltpu.sync_copy(
        acc.at[pl.ds(s * STRIDE, STRIPE)],
        out_hbm.at[pl.ds(c * N_NODES + s * STRIDE, STRIPE)],
    )


def kernel(h, edge_index, W_msg, b_msg, W_node, b_node):
    src = edge_index[0].astype(jnp.int32)
    dst = edge_index[1].astype(jnp.int32)
    a, b = _precompute(h, W_msg, b_msg.reshape(1, D))
    zeros = jnp.zeros((N_NODES, D), jnp.float32)
    mp = _sc_edge_kernel(a, b, src, dst, zeros)
    return _node_update(h, mp[:N_NODES], mp[N_NODES:], W_node,
                        b_node.reshape(1, D))


# speedup vs baseline: 11.3143x; 1.0487x over previous
import functools

import jax
import jax.numpy as jnp
from jax import lax
from jax.experimental import pallas as pl
from jax.experimental.pallas import tpu as pltpu
from jax.experimental.pallas import tpu_sc as plsc

N_NODES = 10000
N_EDGES = 320000
D = 128
LANES = 16

NC, NS = 2, 16
NW = NC * NS
E_PER_W = N_EDGES // NW
K = 40
CHUNKS = E_PER_W // K
STRIPE = 640
STRIDE = 624

NODE_BLK = 1000


def _precompute_body(h_ref, w_ref, b_ref, a_ref, bm_ref):
    hb = h_ref[...]
    w = w_ref[...]
    a_ref[...] = jnp.dot(hb, w[:D], preferred_element_type=jnp.float32)
    bm_ref[...] = (
        jnp.dot(hb, w[D:], preferred_element_type=jnp.float32) + b_ref[...]
    )


def _precompute(h, W_msg, b_msg2d):
    grid = N_NODES // NODE_BLK
    return pl.pallas_call(
        _precompute_body,
        grid=(grid,),
        in_specs=[
            pl.BlockSpec((NODE_BLK, D), lambda i: (i, 0)),
            pl.BlockSpec((2 * D, D), lambda i: (0, 0)),
            pl.BlockSpec((1, D), lambda i: (0, 0)),
        ],
        out_specs=[
            pl.BlockSpec((NODE_BLK, D), lambda i: (i, 0)),
            pl.BlockSpec((NODE_BLK, D), lambda i: (i, 0)),
        ],
        out_shape=[
            jax.ShapeDtypeStruct((N_NODES, D), jnp.float32),
            jax.ShapeDtypeStruct((N_NODES, D), jnp.float32),
        ],
    )(h, W_msg, b_msg2d)


def _node_update_body(h_ref, m0_ref, m1_ref, w_ref, b_ref, o_ref):
    w = w_ref[...]
    m = m0_ref[...] + m1_ref[...]
    acc = (
        jnp.dot(h_ref[...], w[:D], preferred_element_type=jnp.float32)
        + jnp.dot(m, w[D:], preferred_element_type=jnp.float32)
        + b_ref[...]
    )
    o_ref[...] = jnp.maximum(acc, 0.0)


def _node_update(h, mp, W_node, b_node2d):
    grid = N_NODES // NODE_BLK
    return pl.pallas_call(
        _node_update_body,
        grid=(grid,),
        in_specs=[
            pl.BlockSpec((NODE_BLK, D), lambda i: (i, 0)),
            pl.BlockSpec((NODE_BLK, D), lambda i: (i, 0)),
            pl.BlockSpec((NODE_BLK, D), lambda i: (i + grid, 0)),
            pl.BlockSpec((2 * D, D), lambda i: (0, 0)),
            pl.BlockSpec((1, D), lambda i: (0, 0)),
        ],
        out_specs=pl.BlockSpec((NODE_BLK, D), lambda i: (i, 0)),
        out_shape=jax.ShapeDtypeStruct((N_NODES, D), jnp.float32),
    )(h, mp, mp, W_node, b_node2d)


@functools.partial(
    pl.kernel,
    out_type=jax.ShapeDtypeStruct((NC * N_NODES, D), jnp.float32),
    mesh=plsc.VectorSubcoreMesh(core_axis_name="c", subcore_axis_name="s"),
    scratch_types=[
        [pltpu.VMEM((K,), jnp.int32)] * 4,
        [pltpu.VMEM((K,), jnp.int32)] * 4,
        [pltpu.VMEM((K, D), jnp.float32)] * 3,
        [pltpu.VMEM((K, D), jnp.float32)] * 3,
        pltpu.VMEM_SHARED((N_NODES, D), jnp.float32),
        [pltpu.SemaphoreType.DMA] * 4,
        [pltpu.SemaphoreType.DMA] * 3,
        [pltpu.SemaphoreType.DMA] * 3,
        [pltpu.SemaphoreType.DMA] * 3,
    ],
)
def _sc_edge_kernel(a_hbm, b_hbm, src_hbm, dst_hbm, out_hbm,
                    idx_s, idx_d, buf_a, buf_b, acc,
                    sem_i, sem_a, sem_b, sem_s):
    c = lax.axis_index("c")
    s = lax.axis_index("s")
    wid = s * NC + c
    base = wid * E_PER_W

    def zrow(r, rc):
        for g in range(D // LANES):
            buf_a[0][r, pl.ds(g * LANES, LANES)] = jnp.zeros(
                (LANES,), jnp.float32)
        return rc

    lax.fori_loop(0, K, zrow, 0)
    for t in range(STRIPE // K):
        pltpu.sync_copy(buf_a[0], acc.at[pl.ds(s * STRIDE + t * K, K)])
    plsc.subcore_barrier()

    def fire_idx(j, pi):
        off = base + j * K
        pltpu.async_copy(src_hbm.at[pl.ds(off, K)], idx_s[pi], sem_i[pi])
        pltpu.async_copy(dst_hbm.at[pl.ds(off, K)], idx_d[pi], sem_i[pi])

    def wait_idx(pi):
        pltpu.make_async_copy(src_hbm.at[pl.ds(0, K)], idx_s[pi], sem_i[pi]).wait()
        pltpu.make_async_copy(dst_hbm.at[pl.ds(0, K)], idx_d[pi], sem_i[pi]).wait()

    def fire_gather(pr, pi):
        pltpu.async_copy(a_hbm.at[idx_s[pi]], buf_a[pr], sem_a[pr])
        pltpu.async_copy(b_hbm.at[idx_d[pi]], buf_b[pr], sem_b[pr])

    def wait_gather(pr, pi):
        pltpu.make_async_copy(a_hbm.at[idx_s[pi]], buf_a[pr], sem_a[pr]).wait()
        pltpu.make_async_copy(b_hbm.at[idx_d[pi]], buf_b[pr], sem_b[pr]).wait()

    def compute(pr):
        def row_body(r, rc):
            for g in range(D // LANES):
                sl = pl.ds(g * LANES, LANES)
                buf_a[pr][r, sl] = jnp.maximum(
                    buf_a[pr][r, sl] + buf_b[pr][r, sl], 0.0)
            return rc

        lax.fori_loop(0, K, row_body, 0)

    def fire_scatter(pr, pi):
        pltpu.async_copy(buf_a[pr], acc.at[idx_d[pi]], sem_s[pr], add=True)

    def wait_scatter(pr, pi):
        pltpu.make_async_copy(buf_a[pr], acc.at[idx_d[pi]], sem_s[pr]).wait()

    fire_idx(0, 0)
    fire_idx(1, 1)
    fire_idx(2, 2)
    wait_idx(0)
    fire_gather(0, 0)
    wait_idx(1)
    fire_gather(1, 1)

    STEPS_MAIN = 12 * ((CHUNKS - 5) // 12)

    def block_body(t, carry):
        j0 = 12 * t
        for k in range(12):
            j = j0 + k
            jr, ji = k % 3, k % 4
            wait_gather(jr, ji)
            compute(jr)
            fire_scatter(jr, ji)
            if k == 0:
                @pl.when(t > 0)
                def _():
                    wait_scatter(2, 3)
            else:
                wait_scatter((k - 1) % 3, (k - 1) % 4)
            wait_idx((k + 2) % 4)
            fire_gather((k + 2) % 3, (k + 2) % 4)
            fire_idx(j + 3, (k + 3) % 4)
        return carry

    lax.fori_loop(0, STEPS_MAIN // 12, block_body, 0)

    for j in range(STEPS_MAIN, CHUNKS):
        jr, ji = j % 3, j % 4
        wait_gather(jr, ji)
        compute(jr)
        fire_scatter(jr, ji)
        wait_scatter((j - 1) % 3, (j - 1) % 4)
        if j + 2 < CHUNKS:
            wait_idx((j + 2) % 4)
            fire_gather((j + 2) % 3, (j + 2) % 4)
        if j + 3 < CHUNKS:
            fire_idx(j + 3, (j + 3) % 4)

    wait_scatter((CHUNKS - 1) % 3, (CHUNKS - 1) % 4)

    plsc.subcore_barrier()
    pltpu.sync_copy(
        acc.at[pl.ds(s * STRIDE, STRIPE)],
        out_hbm.at[pl.ds(c * N_NODES + s * STRIDE, STRIPE)],
    )


def kernel(h, edge_index, W_msg, b_msg, W_node, b_node):
    src = edge_index[0].astype(jnp.int32)
    dst = edge_index[1].astype(jnp.int32)
    a, b = _precompute(h, W_msg, b_msg.reshape(1, D))
    mp = _sc_edge_kernel(a, b, src, dst)
    return _node_update(h, mp, W_node, b_node.reshape(1, D))


# speedup vs baseline: 12.8129x; 1.1325x over previous
import functools

import jax
import jax.numpy as jnp
from jax import lax
from jax.experimental import pallas as pl
from jax.experimental.pallas import tpu as pltpu
from jax.experimental.pallas import tpu_sc as plsc

N_NODES = 10000
N_EDGES = 320000
D = 128
LANES = 16

NC, NS = 2, 16
NW = NC * NS
E_PER_W = N_EDGES // NW
K = 40
CHUNKS = E_PER_W // K
STRIPE = 640
STRIDE = 624

NODE_BLK = 1000


def _precompute_body(h_ref, w_ref, b_ref, a_ref, bm_ref):
    hb = h_ref[...]
    w = w_ref[...]
    a_ref[...] = jnp.dot(hb, w[:D], preferred_element_type=jnp.float32)
    bm_ref[...] = (
        jnp.dot(hb, w[D:], preferred_element_type=jnp.float32) + b_ref[...]
    )


def _precompute(h, W_msg, b_msg2d):
    grid = N_NODES // NODE_BLK
    return pl.pallas_call(
        _precompute_body,
        grid=(grid,),
        in_specs=[
            pl.BlockSpec((NODE_BLK, D), lambda i: (i, 0)),
            pl.BlockSpec((2 * D, D), lambda i: (0, 0)),
            pl.BlockSpec((1, D), lambda i: (0, 0)),
        ],
        out_specs=[
            pl.BlockSpec((NODE_BLK, D), lambda i: (i, 0)),
            pl.BlockSpec((NODE_BLK, D), lambda i: (i, 0)),
        ],
        out_shape=[
            jax.ShapeDtypeStruct((N_NODES, D), jnp.float32),
            jax.ShapeDtypeStruct((N_NODES, D), jnp.float32),
        ],
    )(h, W_msg, b_msg2d)


def _node_update_body(h_ref, m0_ref, m1_ref, w_ref, b_ref, o_ref):
    w = w_ref[...]
    m = m0_ref[...] + m1_ref[...]
    acc = (
        jnp.dot(h_ref[...], w[:D], preferred_element_type=jnp.float32)
        + jnp.dot(m, w[D:], preferred_element_type=jnp.float32)
        + b_ref[...]
    )
    o_ref[...] = jnp.maximum(acc, 0.0)


def _node_update(h, mp, W_node, b_node2d):
    grid = N_NODES // NODE_BLK
    return pl.pallas_call(
        _node_update_body,
        grid=(grid,),
        in_specs=[
            pl.BlockSpec((NODE_BLK, D), lambda i: (i, 0)),
            pl.BlockSpec((NODE_BLK, D), lambda i: (i, 0)),
            pl.BlockSpec((NODE_BLK, D), lambda i: (i + grid, 0)),
            pl.BlockSpec((2 * D, D), lambda i: (0, 0)),
            pl.BlockSpec((1, D), lambda i: (0, 0)),
        ],
        out_specs=pl.BlockSpec((NODE_BLK, D), lambda i: (i, 0)),
        out_shape=jax.ShapeDtypeStruct((N_NODES, D), jnp.float32),
    )(h, mp, mp, W_node, b_node2d)


@functools.partial(
    pl.kernel,
    out_type=jax.ShapeDtypeStruct((NC * N_NODES, D), jnp.float32),
    mesh=plsc.VectorSubcoreMesh(core_axis_name="c", subcore_axis_name="s"),
    scratch_types=[
        [pltpu.VMEM((K,), jnp.int32)] * 6,
        [pltpu.VMEM((K,), jnp.int32)] * 6,
        [pltpu.VMEM((K, D), jnp.float32)] * 4,
        [pltpu.VMEM((K, D), jnp.float32)] * 4,
        pltpu.VMEM_SHARED((N_NODES, D), jnp.float32),
        [pltpu.SemaphoreType.DMA] * 6,
        [pltpu.SemaphoreType.DMA] * 4,
        [pltpu.SemaphoreType.DMA] * 4,
        [pltpu.SemaphoreType.DMA] * 4,
    ],
)
def _sc_edge_kernel(a_hbm, b_hbm, src_hbm, dst_hbm, out_hbm,
                    idx_s, idx_d, buf_a, buf_b, acc,
                    sem_i, sem_a, sem_b, sem_s):
    c = lax.axis_index("c")
    s = lax.axis_index("s")
    wid = s * NC + c
    base = wid * E_PER_W

    def zrow(r, rc):
        for g in range(D // LANES):
            buf_a[0][r, pl.ds(g * LANES, LANES)] = jnp.zeros(
                (LANES,), jnp.float32)
        return rc

    lax.fori_loop(0, K, zrow, 0)
    for t in range(STRIPE // K):
        pltpu.sync_copy(buf_a[0], acc.at[pl.ds(s * STRIDE + t * K, K)])
    plsc.subcore_barrier()

    def fire_idx(j, pi):
        off = base + j * K
        pltpu.async_copy(src_hbm.at[pl.ds(off, K)], idx_s[pi], sem_i[pi])
        pltpu.async_copy(dst_hbm.at[pl.ds(off, K)], idx_d[pi], sem_i[pi])

    def wait_idx(pi):
        pltpu.make_async_copy(src_hbm.at[pl.ds(0, K)], idx_s[pi], sem_i[pi]).wait()
        pltpu.make_async_copy(dst_hbm.at[pl.ds(0, K)], idx_d[pi], sem_i[pi]).wait()

    def fire_gather(pr, pi):
        pltpu.async_copy(a_hbm.at[idx_s[pi]], buf_a[pr], sem_a[pr])
        pltpu.async_copy(b_hbm.at[idx_d[pi]], buf_b[pr], sem_b[pr])

    def wait_gather(pr, pi):
        pltpu.make_async_copy(a_hbm.at[idx_s[pi]], buf_a[pr], sem_a[pr]).wait()
        pltpu.make_async_copy(b_hbm.at[idx_d[pi]], buf_b[pr], sem_b[pr]).wait()

    def compute(pr):
        def row_body(r, rc):
            for g in range(D // LANES):
                sl = pl.ds(g * LANES, LANES)
                buf_a[pr][r, sl] = jnp.maximum(
                    buf_a[pr][r, sl] + buf_b[pr][r, sl], 0.0)
            return rc

        lax.fori_loop(0, K, row_body, 0)

    def fire_scatter(pr, pi):
        pltpu.async_copy(buf_a[pr], acc.at[idx_d[pi]], sem_s[pr], add=True)

    def wait_scatter(pr, pi):
        pltpu.make_async_copy(buf_a[pr], acc.at[idx_d[pi]], sem_s[pr]).wait()

    fire_idx(0, 0)
    fire_idx(1, 1)
    fire_idx(2, 2)
    fire_idx(3, 3)
    fire_idx(4, 4)
    wait_idx(0)
    fire_gather(0, 0)
    wait_idx(1)
    fire_gather(1, 1)
    wait_idx(2)
    fire_gather(2, 2)

    STEPS_MAIN = 12 * ((CHUNKS - 10) // 12)

    def block_body(t, carry):
        j0 = 12 * t
        for k in range(12):
            j = j0 + k
            jr, ji = k % 4, k % 6
            wait_gather(jr, ji)
            compute(jr)
            if k == 0:
                @pl.when(t > 0)
                def _():
                    wait_scatter(3, 5)
            else:
                wait_scatter((k - 1) % 4, (k - 1) % 6)
            fire_scatter(jr, ji)
            wait_idx((k + 3) % 6)
            fire_gather((k + 3) % 4, (k + 3) % 6)
            fire_idx(j + 5, (k + 5) % 6)
        return carry

    lax.fori_loop(0, STEPS_MAIN // 12, block_body, 0)

    for j in range(STEPS_MAIN, CHUNKS):
        jr, ji = j % 4, j % 6
        wait_gather(jr, ji)
        compute(jr)
        wait_scatter((j - 1) % 4, (j - 1) % 6)
        fire_scatter(jr, ji)
        if j + 3 < CHUNKS:
            wait_idx((j + 3) % 6)
            fire_gather((j + 3) % 4, (j + 3) % 6)
        if j + 5 < CHUNKS:
            fire_idx(j + 5, (j + 5) % 6)

    wait_scatter((CHUNKS - 1) % 4, (CHUNKS - 1) % 6)

    plsc.subcore_barrier()
    pltpu.sync_copy(
        acc.at[pl.ds(s * STRIDE, STRIPE)],
        out_hbm.at[pl.ds(c * N_NODES + s * STRIDE, STRIPE)],
    )


def kernel(h, edge_index, W_msg, b_msg, W_node, b_node):
    src = edge_index[0].astype(jnp.int32)
    dst = edge_index[1].astype(jnp.int32)
    a, b = _precompute(h, W_msg, b_msg.reshape(1, D))
    mp = _sc_edge_kernel(a, b, src, dst)
    return _node_update(h, mp, W_node, b_node.reshape(1, D))


# speedup vs baseline: 13.2410x; 1.0334x over previous
import functools

import jax
import jax.numpy as jnp
from jax import lax
from jax.experimental import pallas as pl
from jax.experimental.pallas import tpu as pltpu
from jax.experimental.pallas import tpu_sc as plsc

N_NODES = 10000
N_EDGES = 320000
D = 128
LANES = 16

NC, NS = 2, 16
NW = NC * NS
E_PER_W = N_EDGES // NW
K = 40
CHUNKS = E_PER_W // K
STRIPE = 640
STRIDE = 624

NODE_BLK = 1000


def _precompute_body(h_ref, w_ref, b_ref, a_ref, bm_ref):
    hb = h_ref[...]
    w = w_ref[...]
    a_ref[...] = jnp.dot(hb, w[:D], preferred_element_type=jnp.float32)
    bm_ref[...] = (
        jnp.dot(hb, w[D:], preferred_element_type=jnp.float32) + b_ref[...]
    )


def _precompute(h, W_msg, b_msg2d):
    grid = N_NODES // NODE_BLK
    return pl.pallas_call(
        _precompute_body,
        grid=(grid,),
        in_specs=[
            pl.BlockSpec((NODE_BLK, D), lambda i: (i, 0)),
            pl.BlockSpec((2 * D, D), lambda i: (0, 0)),
            pl.BlockSpec((1, D), lambda i: (0, 0)),
        ],
        out_specs=[
            pl.BlockSpec((NODE_BLK, D), lambda i: (i, 0)),
            pl.BlockSpec((NODE_BLK, D), lambda i: (i, 0)),
        ],
        out_shape=[
            jax.ShapeDtypeStruct((N_NODES, D), jnp.float32),
            jax.ShapeDtypeStruct((N_NODES, D), jnp.float32),
        ],
    )(h, W_msg, b_msg2d)


def _node_update_body(h_ref, m0_ref, m1_ref, w_ref, b_ref, o_ref):
    w = w_ref[...]
    m = m0_ref[...] + m1_ref[...]
    acc = (
        jnp.dot(h_ref[...], w[:D], preferred_element_type=jnp.float32)
        + jnp.dot(m, w[D:], preferred_element_type=jnp.float32)
        + b_ref[...]
    )
    o_ref[...] = jnp.maximum(acc, 0.0)


def _node_update(h, mp, W_node, b_node2d):
    grid = N_NODES // NODE_BLK
    return pl.pallas_call(
        _node_update_body,
        grid=(grid,),
        in_specs=[
            pl.BlockSpec((NODE_BLK, D), lambda i: (i, 0)),
            pl.BlockSpec((NODE_BLK, D), lambda i: (i, 0)),
            pl.BlockSpec((NODE_BLK, D), lambda i: (i + grid, 0)),
            pl.BlockSpec((2 * D, D), lambda i: (0, 0)),
            pl.BlockSpec((1, D), lambda i: (0, 0)),
        ],
        out_specs=pl.BlockSpec((NODE_BLK, D), lambda i: (i, 0)),
        out_shape=jax.ShapeDtypeStruct((N_NODES, D), jnp.float32),
    )(h, mp, mp, W_node, b_node2d)


@functools.partial(
    pl.kernel,
    out_type=jax.ShapeDtypeStruct((NC * N_NODES, D), jnp.float32),
    mesh=plsc.VectorSubcoreMesh(core_axis_name="c", subcore_axis_name="s"),
    scratch_types=[
        [pltpu.VMEM((K,), jnp.int32)] * 8,
        [pltpu.VMEM((K,), jnp.int32)] * 8,
        [pltpu.VMEM((K, D), jnp.float32)] * 5,
        [pltpu.VMEM((K, D), jnp.float32)] * 4,
        pltpu.VMEM_SHARED((N_NODES, D), jnp.float32),
        [pltpu.SemaphoreType.DMA] * 8,
        [pltpu.SemaphoreType.DMA] * 5,
        [pltpu.SemaphoreType.DMA] * 4,
        [pltpu.SemaphoreType.DMA] * 5,
    ],
)
def _sc_edge_kernel(a_hbm, b_hbm, src_hbm, dst_hbm, out_hbm,
                    idx_s, idx_d, buf_a, buf_b, acc,
                    sem_i, sem_a, sem_b, sem_s):
    c = lax.axis_index("c")
    s = lax.axis_index("s")
    wid = s * NC + c
    base = wid * E_PER_W

    def zrow(r, rc):
        for g in range(D // LANES):
            buf_a[0][r, pl.ds(g * LANES, LANES)] = jnp.zeros(
                (LANES,), jnp.float32)
        return rc

    lax.fori_loop(0, K, zrow, 0)
    for t in range(STRIPE // K):
        pltpu.sync_copy(buf_a[0], acc.at[pl.ds(s * STRIDE + t * K, K)])
    plsc.subcore_barrier()

    def fire_idx(j, pi):
        off = base + j * K
        pltpu.async_copy(src_hbm.at[pl.ds(off, K)], idx_s[pi], sem_i[pi])
        pltpu.async_copy(dst_hbm.at[pl.ds(off, K)], idx_d[pi], sem_i[pi])

    def wait_idx(pi):
        pltpu.make_async_copy(src_hbm.at[pl.ds(0, K)], idx_s[pi], sem_i[pi]).wait()
        pltpu.make_async_copy(dst_hbm.at[pl.ds(0, K)], idx_d[pi], sem_i[pi]).wait()

    def fire_gather(pa, pb, pi):
        pltpu.async_copy(a_hbm.at[idx_s[pi]], buf_a[pa], sem_a[pa])
        pltpu.async_copy(b_hbm.at[idx_d[pi]], buf_b[pb], sem_b[pb])

    def wait_gather(pa, pb, pi):
        pltpu.make_async_copy(a_hbm.at[idx_s[pi]], buf_a[pa], sem_a[pa]).wait()
        pltpu.make_async_copy(b_hbm.at[idx_d[pi]], buf_b[pb], sem_b[pb]).wait()

    def compute(pa, pb):
        def row_body(r, rc):
            for g in range(D // LANES):
                sl = pl.ds(g * LANES, LANES)
                buf_a[pa][r, sl] = jnp.maximum(
                    buf_a[pa][r, sl] + buf_b[pb][r, sl], 0.0)
            return rc

        lax.fori_loop(0, K, row_body, 0)

    def fire_scatter(pa, pi):
        pltpu.async_copy(buf_a[pa], acc.at[idx_d[pi]], sem_s[pa], add=True)

    def wait_scatter(pa, pi):
        pltpu.make_async_copy(buf_a[pa], acc.at[idx_d[pi]], sem_s[pa]).wait()

    for jj in range(6):
        fire_idx(jj, jj)
    for jj in range(4):
        wait_idx(jj)
        fire_gather(jj, jj, jj)

    STEPS_MAIN = 40 * ((CHUNKS - 10) // 40)

    def block_body(t, carry):
        j0 = 40 * t
        for k in range(40):
            j = j0 + k
            pa, pb, pi = k % 5, k % 4, k % 8
            wait_gather(pa, pb, pi)
            compute(pa, pb)
            if k == 0:
                @pl.when(t > 0)
                def _():
                    wait_scatter(4, 7)
            else:
                wait_scatter((k - 1) % 5, (k - 1) % 8)
            fire_scatter(pa, pi)
            wait_idx((k + 4) % 8)
            fire_gather((k + 4) % 5, (k + 4) % 4, (k + 4) % 8)
            fire_idx(j + 6, (k + 6) % 8)
        return carry

    lax.fori_loop(0, STEPS_MAIN // 40, block_body, 0)

    for j in range(STEPS_MAIN, CHUNKS):
        pa, pb, pi = j % 5, j % 4, j % 8
        wait_gather(pa, pb, pi)
        compute(pa, pb)
        wait_scatter((j - 1) % 5, (j - 1) % 8)
        fire_scatter(pa, pi)
        if j + 4 < CHUNKS:
            wait_idx((j + 4) % 8)
            fire_gather((j + 4) % 5, (j + 4) % 4, (j + 4) % 8)
        if j + 6 < CHUNKS:
            fire_idx(j + 6, (j + 6) % 8)

    wait_scatter((CHUNKS - 1) % 5, (CHUNKS - 1) % 8)

    plsc.subcore_barrier()
    pltpu.sync_copy(
        acc.at[pl.ds(s * STRIDE, STRIPE)],
        out_hbm.at[pl.ds(c * N_NODES + s * STRIDE, STRIPE)],
    )


def kernel(h, edge_index, W_msg, b_msg, W_node, b_node):
    src = edge_index[0].astype(jnp.int32)
    dst = edge_index[1].astype(jnp.int32)
    a, b = _precompute(h, W_msg, b_msg.reshape(1, D))
    mp = _sc_edge_kernel(a, b, src, dst)
    return _node_update(h, mp, W_node, b_node.reshape(1, D))


# speedup vs baseline: 13.5587x; 1.0240x over previous
import functools

import jax
import jax.numpy as jnp
from jax import lax
from jax.experimental import pallas as pl
from jax.experimental.pallas import tpu as pltpu
from jax.experimental.pallas import tpu_sc as plsc

N_NODES = 10000
N_EDGES = 320000
D = 128
LANES = 16

NC, NS = 2, 16
NW = NC * NS
E_PER_W = N_EDGES // NW
K = 40
CHUNKS = E_PER_W // K
STRIPE = 640
STRIDE = 624

NODE_BLK = 2000


def _precompute_body(h_ref, w_ref, b_ref, a_ref, bm_ref):
    hb = h_ref[...]
    w = w_ref[...]
    a_ref[...] = jnp.dot(hb, w[:D], preferred_element_type=jnp.float32)
    bm_ref[...] = (
        jnp.dot(hb, w[D:], preferred_element_type=jnp.float32) + b_ref[...]
    )


def _precompute(h, W_msg, b_msg2d):
    grid = N_NODES // NODE_BLK
    return pl.pallas_call(
        _precompute_body,
        grid=(grid,),
        in_specs=[
            pl.BlockSpec((NODE_BLK, D), lambda i: (i, 0)),
            pl.BlockSpec((2 * D, D), lambda i: (0, 0)),
            pl.BlockSpec((1, D), lambda i: (0, 0)),
        ],
        out_specs=[
            pl.BlockSpec((NODE_BLK, D), lambda i: (i, 0)),
            pl.BlockSpec((NODE_BLK, D), lambda i: (i, 0)),
        ],
        out_shape=[
            jax.ShapeDtypeStruct((N_NODES, D), jnp.float32),
            jax.ShapeDtypeStruct((N_NODES, D), jnp.float32),
        ],
    )(h, W_msg, b_msg2d)


def _node_update_body(h_ref, m0_ref, m1_ref, w_ref, b_ref, o_ref):
    w = w_ref[...]
    m = m0_ref[...] + m1_ref[...]
    acc = (
        jnp.dot(h_ref[...], w[:D], preferred_element_type=jnp.float32)
        + jnp.dot(m, w[D:], preferred_element_type=jnp.float32)
        + b_ref[...]
    )
    o_ref[...] = jnp.maximum(acc, 0.0)


def _node_update(h, mp, W_node, b_node2d):
    grid = N_NODES // NODE_BLK
    return pl.pallas_call(
        _node_update_body,
        grid=(grid,),
        in_specs=[
            pl.BlockSpec((NODE_BLK, D), lambda i: (i, 0)),
            pl.BlockSpec((NODE_BLK, D), lambda i: (i, 0)),
            pl.BlockSpec((NODE_BLK, D), lambda i: (i + grid, 0)),
            pl.BlockSpec((2 * D, D), lambda i: (0, 0)),
            pl.BlockSpec((1, D), lambda i: (0, 0)),
        ],
        out_specs=pl.BlockSpec((NODE_BLK, D), lambda i: (i, 0)),
        out_shape=jax.ShapeDtypeStruct((N_NODES, D), jnp.float32),
    )(h, mp, mp, W_node, b_node2d)


@functools.partial(
    pl.kernel,
    out_type=jax.ShapeDtypeStruct((NC * N_NODES, D), jnp.float32),
    mesh=plsc.VectorSubcoreMesh(core_axis_name="c", subcore_axis_name="s"),
    scratch_types=[
        [pltpu.VMEM((K,), jnp.int32)] * 8,
        [pltpu.VMEM((K,), jnp.int32)] * 8,
        [pltpu.VMEM((K, D), jnp.float32)] * 5,
        [pltpu.VMEM((K, D), jnp.float32)] * 4,
        pltpu.VMEM_SHARED((N_NODES, D), jnp.float32),
        [pltpu.SemaphoreType.DMA] * 8,
        [pltpu.SemaphoreType.DMA] * 5,
        [pltpu.SemaphoreType.DMA] * 4,
        [pltpu.SemaphoreType.DMA] * 5,
    ],
)
def _sc_edge_kernel(a_hbm, b_hbm, src_hbm, dst_hbm, out_hbm,
                    idx_s, idx_d, buf_a, buf_b, acc,
                    sem_i, sem_a, sem_b, sem_s):
    c = lax.axis_index("c")
    s = lax.axis_index("s")
    wid = s * NC + c
    base = wid * E_PER_W

    def zrow(r, rc):
        for g in range(D // LANES):
            buf_a[0][r, pl.ds(g * LANES, LANES)] = jnp.zeros(
                (LANES,), jnp.float32)
        return rc

    lax.fori_loop(0, K, zrow, 0)
    for t in range(STRIPE // K):
        pltpu.sync_copy(buf_a[0], acc.at[pl.ds(s * STRIDE + t * K, K)])
    plsc.subcore_barrier()

    def fire_idx(j, pi):
        off = base + j * K
        pltpu.async_copy(src_hbm.at[pl.ds(off, K)], idx_s[pi], sem_i[pi])
        pltpu.async_copy(dst_hbm.at[pl.ds(off, K)], idx_d[pi], sem_i[pi])

    def wait_idx(pi):
        pltpu.make_async_copy(src_hbm.at[pl.ds(0, K)], idx_s[pi], sem_i[pi]).wait()
        pltpu.make_async_copy(dst_hbm.at[pl.ds(0, K)], idx_d[pi], sem_i[pi]).wait()

    def fire_gather(pa, pb, pi):
        pltpu.async_copy(a_hbm.at[idx_s[pi]], buf_a[pa], sem_a[pa])
        pltpu.async_copy(b_hbm.at[idx_d[pi]], buf_b[pb], sem_b[pb])

    def wait_gather(pa, pb, pi):
        pltpu.make_async_copy(a_hbm.at[idx_s[pi]], buf_a[pa], sem_a[pa]).wait()
        pltpu.make_async_copy(b_hbm.at[idx_d[pi]], buf_b[pb], sem_b[pb]).wait()

    def compute(pa, pb):
        def row_body(r2, rc):
            for u in range(2):
                r = 2 * r2 + u
                for g in range(D // LANES):
                    sl = pl.ds(g * LANES, LANES)
                    buf_a[pa][r, sl] = jnp.maximum(
                        buf_a[pa][r, sl] + buf_b[pb][r, sl], 0.0)
            return rc

        lax.fori_loop(0, K // 2, row_body, 0)

    def fire_scatter(pa, pi):
        pltpu.async_copy(buf_a[pa], acc.at[idx_d[pi]], sem_s[pa], add=True)

    def wait_scatter(pa, pi):
        pltpu.make_async_copy(buf_a[pa], acc.at[idx_d[pi]], sem_s[pa]).wait()

    for jj in range(6):
        fire_idx(jj, jj)
    for jj in range(4):
        wait_idx(jj)
        fire_gather(jj, jj, jj)

    STEPS_MAIN = 40 * ((CHUNKS - 10) // 40)

    def block_body(t, carry):
        j0 = 40 * t
        for k in range(40):
            j = j0 + k
            pa, pb, pi = k % 5, k % 4, k % 8
            wait_gather(pa, pb, pi)
            compute(pa, pb)
            if k == 0:
                @pl.when(t > 0)
                def _():
                    wait_scatter(4, 7)
            else:
                wait_scatter((k - 1) % 5, (k - 1) % 8)
            fire_scatter(pa, pi)
            wait_idx((k + 4) % 8)
            fire_gather((k + 4) % 5, (k + 4) % 4, (k + 4) % 8)
            fire_idx(j + 6, (k + 6) % 8)
        return carry

    lax.fori_loop(0, STEPS_MAIN // 40, block_body, 0)

    for j in range(STEPS_MAIN, CHUNKS):
        pa, pb, pi = j % 5, j % 4, j % 8
        wait_gather(pa, pb, pi)
        compute(pa, pb)
        wait_scatter((j - 1) % 5, (j - 1) % 8)
        fire_scatter(pa, pi)
        if j + 4 < CHUNKS:
            wait_idx((j + 4) % 8)
            fire_gather((j + 4) % 5, (j + 4) % 4, (j + 4) % 8)
        if j + 6 < CHUNKS:
            fire_idx(j + 6, (j + 6) % 8)

    wait_scatter((CHUNKS - 1) % 5, (CHUNKS - 1) % 8)

    plsc.subcore_barrier()
    pltpu.sync_copy(
        acc.at[pl.ds(s * STRIDE, STRIPE)],
        out_hbm.at[pl.ds(c * N_NODES + s * STRIDE, STRIPE)],
    )


def kernel(h, edge_index, W_msg, b_msg, W_node, b_node):
    src = edge_index[0].astype(jnp.int32)
    dst = edge_index[1].astype(jnp.int32)
    a, b = _precompute(h, W_msg, b_msg.reshape(1, D))
    mp = _sc_edge_kernel(a, b, src, dst)
    return _node_update(h, mp, W_node, b_node.reshape(1, D))


# speedup vs baseline: 13.6131x; 1.0040x over previous
import functools

import jax
import jax.numpy as jnp
from jax import lax
from jax.experimental import pallas as pl
from jax.experimental.pallas import tpu as pltpu
from jax.experimental.pallas import tpu_sc as plsc

N_NODES = 10000
N_EDGES = 320000
D = 128
LANES = 16

NC, NS = 2, 16
NW = NC * NS
E_PER_W = N_EDGES // NW
K = 40
CHUNKS = E_PER_W // K
STRIPE = 640
STRIDE = 624

NODE_BLK = 2000


def _precompute_body(h_ref, w_ref, b_ref, a_ref, bm_ref):
    hb = h_ref[...]
    w = w_ref[...]
    a_ref[...] = jnp.dot(hb, w[:D], preferred_element_type=jnp.float32)
    bm_ref[...] = (
        jnp.dot(hb, w[D:], preferred_element_type=jnp.float32) + b_ref[...]
    )


def _precompute(h, W_msg, b_msg2d):
    grid = N_NODES // NODE_BLK
    return pl.pallas_call(
        _precompute_body,
        grid=(grid,),
        in_specs=[
            pl.BlockSpec((NODE_BLK, D), lambda i: (i, 0)),
            pl.BlockSpec((2 * D, D), lambda i: (0, 0)),
            pl.BlockSpec((1, D), lambda i: (0, 0)),
        ],
        out_specs=[
            pl.BlockSpec((NODE_BLK, D), lambda i: (i, 0)),
            pl.BlockSpec((NODE_BLK, D), lambda i: (i, 0)),
        ],
        out_shape=[
            jax.ShapeDtypeStruct((N_NODES, D), jnp.float32),
            jax.ShapeDtypeStruct((N_NODES, D), jnp.float32),
        ],
    )(h, W_msg, b_msg2d)


def _node_update_body(h_ref, m0_ref, m1_ref, w_ref, b_ref, o_ref):
    w = w_ref[...]
    m = m0_ref[...] + m1_ref[...]
    acc = (
        jnp.dot(h_ref[...], w[:D], preferred_element_type=jnp.float32)
        + jnp.dot(m, w[D:], preferred_element_type=jnp.float32)
        + b_ref[...]
    )
    o_ref[...] = jnp.maximum(acc, 0.0)


def _node_update(h, mp, W_node, b_node2d):
    grid = N_NODES // NODE_BLK
    return pl.pallas_call(
        _node_update_body,
        grid=(grid,),
        in_specs=[
            pl.BlockSpec((NODE_BLK, D), lambda i: (i, 0)),
            pl.BlockSpec((NODE_BLK, D), lambda i: (i, 0)),
            pl.BlockSpec((NODE_BLK, D), lambda i: (i + grid, 0)),
            pl.BlockSpec((2 * D, D), lambda i: (0, 0)),
            pl.BlockSpec((1, D), lambda i: (0, 0)),
        ],
        out_specs=pl.BlockSpec((NODE_BLK, D), lambda i: (i, 0)),
        out_shape=jax.ShapeDtypeStruct((N_NODES, D), jnp.float32),
    )(h, mp, mp, W_node, b_node2d)


@functools.partial(
    pl.kernel,
    out_type=jax.ShapeDtypeStruct((NC * N_NODES, D), jnp.float32),
    mesh=plsc.VectorSubcoreMesh(core_axis_name="c", subcore_axis_name="s"),
    scratch_types=[
        [pltpu.VMEM((K,), jnp.int32)] * 8,
        [pltpu.VMEM((K,), jnp.int32)] * 8,
        [pltpu.VMEM((K, D), jnp.float32)] * 5,
        [pltpu.VMEM((K, D), jnp.float32)] * 4,
        pltpu.VMEM_SHARED((N_NODES, D), jnp.float32),
        [pltpu.SemaphoreType.DMA] * 8,
        [pltpu.SemaphoreType.DMA] * 5,
        [pltpu.SemaphoreType.DMA] * 4,
        [pltpu.SemaphoreType.DMA] * 5,
    ],
)
def _sc_edge_kernel(a_hbm, b_hbm, src_hbm, dst_hbm, out_hbm,
                    idx_s, idx_d, buf_a, buf_b, acc,
                    sem_i, sem_a, sem_b, sem_s):
    c = lax.axis_index("c")
    s = lax.axis_index("s")
    wid = s * NC + c
    base = wid * E_PER_W

    def zrow(r, rc):
        for g in range(D // LANES):
            buf_a[0][r, pl.ds(g * LANES, LANES)] = jnp.zeros(
                (LANES,), jnp.float32)
        return rc

    lax.fori_loop(0, K, zrow, 0)
    for t in range(STRIPE // K):
        pltpu.async_copy(buf_a[0], acc.at[pl.ds(s * STRIDE + t * K, K)],
                         sem_s[0])
    for t in range(STRIPE // K):
        pltpu.make_async_copy(buf_a[0], acc.at[pl.ds(s * STRIDE + t * K, K)],
                              sem_s[0]).wait()
    plsc.subcore_barrier()

    def fire_idx(j, pi):
        off = base + j * K
        pltpu.async_copy(src_hbm.at[pl.ds(off, K)], idx_s[pi], sem_i[pi])
        pltpu.async_copy(dst_hbm.at[pl.ds(off, K)], idx_d[pi], sem_i[pi])

    def wait_idx(pi):
        pltpu.make_async_copy(src_hbm.at[pl.ds(0, K)], idx_s[pi], sem_i[pi]).wait()
        pltpu.make_async_copy(dst_hbm.at[pl.ds(0, K)], idx_d[pi], sem_i[pi]).wait()

    def fire_gather(pa, pb, pi):
        pltpu.async_copy(a_hbm.at[idx_s[pi]], buf_a[pa], sem_a[pa])
        pltpu.async_copy(b_hbm.at[idx_d[pi]], buf_b[pb], sem_b[pb])

    def wait_gather(pa, pb, pi):
        pltpu.make_async_copy(a_hbm.at[idx_s[pi]], buf_a[pa], sem_a[pa]).wait()
        pltpu.make_async_copy(b_hbm.at[idx_d[pi]], buf_b[pb], sem_b[pb]).wait()

    def compute(pa, pb):
        def row_body(r2, rc):
            for u in range(2):
                r = 2 * r2 + u
                for g in range(D // LANES):
                    sl = pl.ds(g * LANES, LANES)
                    buf_a[pa][r, sl] = jnp.maximum(
                        buf_a[pa][r, sl] + buf_b[pb][r, sl], 0.0)
            return rc

        lax.fori_loop(0, K // 2, row_body, 0)

    def fire_scatter(pa, pi):
        pltpu.async_copy(buf_a[pa], acc.at[idx_d[pi]], sem_s[pa], add=True)

    def wait_scatter(pa, pi):
        pltpu.make_async_copy(buf_a[pa], acc.at[idx_d[pi]], sem_s[pa]).wait()

    for jj in range(6):
        fire_idx(jj, jj)
    for jj in range(4):
        wait_idx(jj)
        fire_gather(jj, jj, jj)

    STEPS_MAIN = 40 * ((CHUNKS - 10) // 40)

    def block_body(t, carry):
        j0 = 40 * t
        for k in range(40):
            j = j0 + k
            pa, pb, pi = k % 5, k % 4, k % 8
            wait_gather(pa, pb, pi)
            compute(pa, pb)
            if k == 0:
                @pl.when(t > 0)
                def _():
                    wait_scatter(4, 7)
            else:
                wait_scatter((k - 1) % 5, (k - 1) % 8)
            fire_scatter(pa, pi)
            wait_idx((k + 4) % 8)
            fire_gather((k + 4) % 5, (k + 4) % 4, (k + 4) % 8)
            fire_idx(j + 6, (k + 6) % 8)
        return carry

    lax.fori_loop(0, STEPS_MAIN // 40, block_body, 0)

    for j in range(STEPS_MAIN, CHUNKS):
        pa, pb, pi = j % 5, j % 4, j % 8
        wait_gather(pa, pb, pi)
        compute(pa, pb)
        wait_scatter((j - 1) % 5, (j - 1) % 8)
        fire_scatter(pa, pi)
        if j + 4 < CHUNKS:
            wait_idx((j + 4) % 8)
            fire_gather((j + 4) % 5, (j + 4) % 4, (j + 4) % 8)
        if j + 6 < CHUNKS:
            fire_idx(j + 6, (j + 6) % 8)

    wait_scatter((CHUNKS - 1) % 5, (CHUNKS - 1) % 8)

    plsc.subcore_barrier()
    pltpu.sync_copy(
        acc.at[pl.ds(s * STRIDE, STRIPE)],
        out_hbm.at[pl.ds(c * N_NODES + s * STRIDE, STRIPE)],
    )


def kernel(h, edge_index, W_msg, b_msg, W_node, b_node):
    src = edge_index[0].astype(jnp.int32)
    dst = edge_index[1].astype(jnp.int32)
    a, b = _precompute(h, W_msg, b_msg.reshape(1, D))
    mp = _sc_edge_kernel(a, b, src, dst)
    return _node_update(h, mp, W_node, b_node.reshape(1, D))
